# baseline (speedup 1.0000x reference)
import numpy as np
import jax
import jax.numpy as jnp
from jax import lax
from jax.experimental import pallas as pl
from jax.experimental.pallas import tpu as pltpu
from jax.experimental.pallas import tpu_sc as plsc

_VOLUME = 8
_C = 16
_ROW = _VOLUME * _C
_NC = 2
_NS = 16
_NW = _NC * _NS
_CHUNK_ROWS = 64
_IN_CHUNK = _CHUNK_ROWS * _VOLUME


def _sc_body(x_hbm, o_hbm, in_v, out_v):
    wid = lax.axis_index("s") * _NC + lax.axis_index("c")
    n_rows = o_hbm.shape[0]
    rows_per_w = n_rows // _NW
    n_chunks = rows_per_w // _CHUNK_ROWS
    base_row = wid * rows_per_w

    lane = lax.iota(jnp.int32, 16)
    src_rows = []
    dsts = []
    for k in range(_VOLUME):
        i = ((lane >> 1) + k) & (_VOLUME - 1)
        src_rows.append(i)
        dsts.append(lane * _VOLUME + i)
    zero16 = lane & 0

    def chunk_body(g, _):
        row0 = base_row + g * _CHUNK_ROWS
        pltpu.sync_copy(x_hbm.at[pl.ds(row0 * _VOLUME, _IN_CHUNK), :], in_v)

        def row_body(n, nvec):
            nbase = nvec * _VOLUME
            for k in range(_VOLUME):
                vals = plsc.load_gather(in_v, [nbase + src_rows[k], lane])
                plsc.store_scatter(out_v, [nvec, dsts[k]], vals)
            return nvec + 1

        lax.fori_loop(0, _CHUNK_ROWS, row_body, zero16, unroll=2)
        pltpu.sync_copy(out_v, o_hbm.at[pl.ds(row0, _CHUNK_ROWS), :])
        return 0

    lax.fori_loop(0, n_chunks, chunk_body, 0)


def kernel(features, original_indices):
    n_rows = features.shape[0] // _VOLUME
    mesh = plsc.VectorSubcoreMesh(core_axis_name="c", subcore_axis_name="s")
    out = pl.kernel(
        _sc_body,
        out_type=jax.ShapeDtypeStruct((n_rows, _ROW), jnp.float32),
        mesh=mesh,
        compiler_params=pltpu.CompilerParams(needs_layout_passes=False),
        scratch_types=[
            pltpu.VMEM((_IN_CHUNK, _C), jnp.float32),
            pltpu.VMEM((_CHUNK_ROWS, _ROW), jnp.float32),
        ],
    )(features)
    return out, original_indices


# speedup vs baseline: 1.0703x; 1.0703x over previous
import numpy as np
import jax
import jax.numpy as jnp
from jax import lax
from jax.experimental import pallas as pl
from jax.experimental.pallas import tpu as pltpu
from jax.experimental.pallas import tpu_sc as plsc

_VOLUME = 8
_C = 16
_ROW = _VOLUME * _C
_NC = 2
_NS = 16
_NW = _NC * _NS
_CHUNK_ROWS = 128
_CHUNK_WORDS = _CHUNK_ROWS * _ROW


def _sc_body(x_hbm, o_hbm, in_v, out_v):
    wid = lax.axis_index("s") * _NC + lax.axis_index("c")
    n_rows = o_hbm.shape[0]
    rows_per_w = n_rows // _NW
    n_chunks = rows_per_w // _CHUNK_ROWS
    base_row = wid * rows_per_w

    lane = lax.iota(jnp.int32, 16)
    srcs = []
    dsts = []
    for k in range(_VOLUME):
        i = ((lane >> 1) + k) & (_VOLUME - 1)
        srcs.append(i * _C + lane)
        dsts.append(lane * _VOLUME + i)
    zero16 = lane & 0

    def chunk_body(g, _):
        row0 = base_row + g * _CHUNK_ROWS
        pltpu.sync_copy(x_hbm.at[pl.ds(row0 * _ROW, _CHUNK_WORDS)], in_v)

        def row_body(n, nvec):
            nbase = nvec * _ROW
            for k in range(_VOLUME):
                vals = plsc.load_gather(in_v, [nbase + srcs[k]])
                plsc.store_scatter(out_v, [nvec, dsts[k]], vals)
            return nvec + 1

        lax.fori_loop(0, _CHUNK_ROWS, row_body, zero16, unroll=2)
        pltpu.sync_copy(out_v, o_hbm.at[pl.ds(row0, _CHUNK_ROWS), :])
        return 0

    lax.fori_loop(0, n_chunks, chunk_body, 0)


def kernel(features, original_indices):
    n_rows = features.shape[0] // _VOLUME
    x = features.reshape(n_rows * _ROW)
    mesh = plsc.VectorSubcoreMesh(core_axis_name="c", subcore_axis_name="s")
    out = pl.kernel(
        _sc_body,
        out_type=jax.ShapeDtypeStruct((n_rows, _ROW), jnp.float32),
        mesh=mesh,
        compiler_params=pltpu.CompilerParams(needs_layout_passes=False),
        scratch_types=[
            pltpu.VMEM((_CHUNK_WORDS,), jnp.float32),
            pltpu.VMEM((_CHUNK_ROWS, _ROW), jnp.float32),
        ],
    )(x)
    return out, original_indices


# speedup vs baseline: 1.4263x; 1.3326x over previous
import numpy as np
import jax
import jax.numpy as jnp
from jax import lax
from jax.experimental import pallas as pl
from jax.experimental.pallas import tpu as pltpu
from jax.experimental.pallas import tpu_sc as plsc

_VOLUME = 8
_C = 16
_ROW = _VOLUME * _C
_NC = 2
_NS = 16
_NW = _NC * _NS
_CHUNK_ROWS = 128
_CHUNK_WORDS = _CHUNK_ROWS * _ROW


def _sc_body(x_hbm, o_hbm, in0, in1, out0, out1, si0, si1, so0, so1):
    wid = lax.axis_index("s") * _NC + lax.axis_index("c")
    n_rows = o_hbm.shape[0]
    rows_per_w = n_rows // _NW
    n_chunks = rows_per_w // _CHUNK_ROWS
    base_row = wid * rows_per_w

    lane = lax.iota(jnp.int32, 16)
    srcs = []
    dsts = []
    for k in range(_VOLUME):
        i = ((lane >> 1) + k) & (_VOLUME - 1)
        srcs.append(i * _C + lane)
        dsts.append(lane * _VOLUME + i)
    zero16 = lane & 0

    def in_slice(g):
        return x_hbm.at[pl.ds((base_row + g * _CHUNK_ROWS) * _ROW, _CHUNK_WORDS)]

    def out_slice(g):
        return o_hbm.at[pl.ds(base_row + g * _CHUNK_ROWS, _CHUNK_ROWS), :]

    def permute(in_b, out_b):
        @plsc.parallel_loop(0, _CHUNK_ROWS, carry=zero16)
        def _(n, nvec):
            nbase = nvec * _ROW
            for k in range(_VOLUME):
                vals = plsc.load_gather(in_b, [nbase + srcs[k]])
                plsc.store_scatter(out_b, [nvec, dsts[k]], vals)
            return nvec + 1

    def slot(g, in_b, out_b, si, so, wait_out, load_next):
        pltpu.make_async_copy(in_slice(g), in_b, si).wait()
        if wait_out:
            pltpu.make_async_copy(out_b, out_slice(g), so).wait()
        permute(in_b, out_b)
        pltpu.async_copy(out_b, out_slice(g), so)
        if load_next:
            pltpu.async_copy(in_slice(g + 2), in_b, si)

    pltpu.async_copy(in_slice(0), in0, si0)
    pltpu.async_copy(in_slice(1), in1, si1)
    slot(0, in0, out0, si0, so0, wait_out=False, load_next=True)
    slot(1, in1, out1, si1, so1, wait_out=False, load_next=True)

    def pair_body(g2, _):
        g = g2 * 2
        slot(g, in0, out0, si0, so0, wait_out=True, load_next=True)
        slot(g + 1, in1, out1, si1, so1, wait_out=True, load_next=True)
        return 0

    lax.fori_loop(1, n_chunks // 2 - 1, pair_body, 0)

    slot(n_chunks - 2, in0, out0, si0, so0, wait_out=True, load_next=False)
    slot(n_chunks - 1, in1, out1, si1, so1, wait_out=True, load_next=False)
    pltpu.make_async_copy(out0, out_slice(n_chunks - 2), so0).wait()
    pltpu.make_async_copy(out1, out_slice(n_chunks - 1), so1).wait()


def kernel(features, original_indices):
    n_rows = features.shape[0] // _VOLUME
    x = features.reshape(n_rows * _ROW)
    mesh = plsc.VectorSubcoreMesh(core_axis_name="c", subcore_axis_name="s")
    out = pl.kernel(
        _sc_body,
        out_type=jax.ShapeDtypeStruct((n_rows, _ROW), jnp.float32),
        mesh=mesh,
        compiler_params=pltpu.CompilerParams(needs_layout_passes=False),
        scratch_types=[
            pltpu.VMEM((_CHUNK_WORDS,), jnp.float32),
            pltpu.VMEM((_CHUNK_WORDS,), jnp.float32),
            pltpu.VMEM((_CHUNK_ROWS, _ROW), jnp.float32),
            pltpu.VMEM((_CHUNK_ROWS, _ROW), jnp.float32),
            pltpu.SemaphoreType.DMA,
            pltpu.SemaphoreType.DMA,
            pltpu.SemaphoreType.DMA,
            pltpu.SemaphoreType.DMA,
        ],
    )(x)
    return out, original_indices
